# Initial kernel scaffold; baseline (speedup 1.0000x reference)
import jax
import jax.numpy as jnp
from jax.experimental import pallas as pl


def kernel(x, eigvals, eigvecs):
    raise NotImplementedError("write your pallas kernel here")


# speedup vs baseline: 108.7145x; 108.7145x over previous
import functools

import jax
import jax.numpy as jnp
from jax.experimental import pallas as pl

_TOPK = 512
_RB = 256


def _select_body(x_ref, ev_ref, evals_ref, o_ref, *, k):
    ev = ev_ref[0]
    h = jax.lax.dot_general(
        x_ref[...], ev,
        dimension_numbers=(((1,), (1,)), ((), ())),
        preferred_element_type=jnp.float32,
    )
    hv = evals_ref[0, 0][None, :] * h * h
    abits = jax.lax.bitcast_convert_type(jnp.abs(hv), jnp.int32)
    t = jnp.zeros((abits.shape[0], 1), jnp.int32)
    for bit in range(30, -1, -1):
        cand = t | (1 << bit)
        cnt = jnp.sum((abits >= cand).astype(jnp.int32), axis=1,
                      keepdims=True)
        t = jnp.where(cnt >= k, cand, t)
    o_ref[...] = jnp.where(abits >= t, hv, 0.0)


def kernel(x, eigvals, eigvecs):
    B, D = x.shape
    C, EIG, _ = eigvecs.shape
    k = min(_TOPK, EIG)
    rb = min(_RB, B)
    grid = (C, B // rb)
    out = pl.pallas_call(
        functools.partial(_select_body, k=k),
        grid=grid,
        in_specs=[
            pl.BlockSpec((rb, D), lambda c, b: (b, 0)),
            pl.BlockSpec((1, EIG, D), lambda c, b: (c, 0, 0)),
            pl.BlockSpec((1, 1, EIG), lambda c, b: (c, 0, 0)),
        ],
        out_specs=pl.BlockSpec((rb, EIG), lambda c, b: (b, c)),
        out_shape=jax.ShapeDtypeStruct((B, C * EIG), jnp.float32),
    )(x, eigvecs, eigvals.reshape(C, 1, EIG))
    return out.reshape(B, C, EIG)


# speedup vs baseline: 150.6786x; 1.3860x over previous
import functools

import jax
import jax.numpy as jnp
from jax.experimental import pallas as pl

_TOPK = 512
_RB = 256


def _select_body(x_ref, ev_ref, evals_ref, o_ref, *, k):
    ev = ev_ref[0]
    h = jax.lax.dot_general(
        x_ref[...], ev,
        dimension_numbers=(((1,), (1,)), ((), ())),
        preferred_element_type=jnp.float32,
    )
    hv = evals_ref[0, 0][None, :] * h * h
    abits = jax.lax.bitcast_convert_type(jnp.abs(hv), jnp.int32)
    t = jnp.zeros((abits.shape[0], 1), jnp.int32)
    for bit in range(30, 11, -1):
        cand = t | (1 << bit)
        cnt = jnp.sum((abits >= cand).astype(jnp.int32), axis=1,
                      keepdims=True)
        t = jnp.where(cnt >= k, cand, t)
    o_ref[...] = jnp.where(abits >= t, hv, 0.0)


def kernel(x, eigvals, eigvecs):
    B, D = x.shape
    C, EIG, _ = eigvecs.shape
    k = min(_TOPK, EIG)
    rb = min(_RB, B)
    grid = (C, B // rb)
    out = pl.pallas_call(
        functools.partial(_select_body, k=k),
        grid=grid,
        in_specs=[
            pl.BlockSpec((rb, D), lambda c, b: (b, 0)),
            pl.BlockSpec((1, EIG, D), lambda c, b: (c, 0, 0)),
            pl.BlockSpec((1, 1, EIG), lambda c, b: (c, 0, 0)),
        ],
        out_specs=pl.BlockSpec((rb, EIG), lambda c, b: (b, c)),
        out_shape=jax.ShapeDtypeStruct((B, C * EIG), jnp.float32),
    )(x, eigvecs, eigvals.reshape(C, 1, EIG))
    return out.reshape(B, C, EIG)


# speedup vs baseline: 168.8586x; 1.1207x over previous
import functools

import jax
import jax.numpy as jnp
from jax.experimental import pallas as pl

_TOPK = 512
_RB = 256


def _select_body(x_ref, ev_ref, evals_ref, o_ref, *, k):
    ev = ev_ref[0]
    h = jax.lax.dot_general(
        x_ref[...], ev,
        dimension_numbers=(((1,), (1,)), ((), ())),
        preferred_element_type=jnp.float32,
    )
    hv = evals_ref[0, 0][None, :] * h * h
    abits = jax.lax.bitcast_convert_type(jnp.abs(hv), jnp.int32)
    rows = abits.shape[0]
    hi = (abits >> 16).astype(jnp.int16)
    lanes = 128
    nchunk = hi.shape[1] // lanes

    def count16(ind16):
        acc = ind16[:, :lanes]
        for j in range(1, nchunk):
            acc = acc + ind16[:, j * lanes:(j + 1) * lanes]
        return jnp.sum(acc.astype(jnp.int32), axis=1, keepdims=True)

    t_hi = jnp.zeros((rows, 1), jnp.int32)
    for bit in range(14, -1, -1):
        cand = t_hi | (1 << bit)
        ind = (hi >= cand.astype(jnp.int16)).astype(jnp.int16)
        t_hi = jnp.where(count16(ind) >= k, cand, t_hi)
    t_hi16 = t_hi.astype(jnp.int16)
    in_band = hi == t_hi16
    n_above = count16((hi > t_hi16).astype(jnp.int16))
    m = k - n_above
    lo_s = abits.astype(jnp.int16) ^ jnp.int16(-32768)
    key = jnp.where(in_band, lo_s, jnp.int16(-32768))
    t_lo = jnp.zeros((rows, 1), jnp.int32)
    for bit in range(15, 11, -1):
        cand = t_lo | (1 << bit)
        cand_s16 = (cand - 32768).astype(jnp.int16)
        ind = (key >= cand_s16).astype(jnp.int16)
        t_lo = jnp.where(count16(ind) >= m, cand, t_lo)
    t = (t_hi << 16) | t_lo
    o_ref[...] = jnp.where(abits >= t, hv, 0.0)


def kernel(x, eigvals, eigvecs):
    B, D = x.shape
    C, EIG, _ = eigvecs.shape
    k = min(_TOPK, EIG)
    rb = min(_RB, B)
    grid = (C, B // rb)
    out = pl.pallas_call(
        functools.partial(_select_body, k=k),
        grid=grid,
        in_specs=[
            pl.BlockSpec((rb, D), lambda c, b: (b, 0)),
            pl.BlockSpec((1, EIG, D), lambda c, b: (c, 0, 0)),
            pl.BlockSpec((1, 1, EIG), lambda c, b: (c, 0, 0)),
        ],
        out_specs=pl.BlockSpec((rb, EIG), lambda c, b: (b, c)),
        out_shape=jax.ShapeDtypeStruct((B, C * EIG), jnp.float32),
    )(x, eigvecs, eigvals.reshape(C, 1, EIG))
    return out.reshape(B, C, EIG)


# speedup vs baseline: 178.9144x; 1.0596x over previous
import functools

import jax
import jax.numpy as jnp
from jax.experimental import pallas as pl

_TOPK = 512
_RB = 512


def _select_body(x_ref, ev_ref, evals_ref, o_ref, *, k):
    ev = ev_ref[0]
    h = jax.lax.dot_general(
        x_ref[...], ev,
        dimension_numbers=(((1,), (1,)), ((), ())),
        preferred_element_type=jnp.float32,
    )
    hv = evals_ref[0, 0][None, :] * h * h
    abits = jax.lax.bitcast_convert_type(jnp.abs(hv), jnp.int32)
    rows = abits.shape[0]
    hi = (abits >> 16).astype(jnp.int16)
    lanes = 128
    nchunk = hi.shape[1] // lanes

    def count16(ind16):
        acc = ind16[:, :lanes]
        for j in range(1, nchunk):
            acc = acc + ind16[:, j * lanes:(j + 1) * lanes]
        return jnp.sum(acc.astype(jnp.int32), axis=1, keepdims=True)

    t_hi = jnp.zeros((rows, 1), jnp.int32)
    for bit in range(14, -1, -1):
        cand = t_hi | (1 << bit)
        ind = (hi >= cand.astype(jnp.int16)).astype(jnp.int16)
        t_hi = jnp.where(count16(ind) >= k, cand, t_hi)
    t_hi16 = t_hi.astype(jnp.int16)
    in_band = hi == t_hi16
    n_above = count16((hi > t_hi16).astype(jnp.int16))
    m = k - n_above
    lo_s = abits.astype(jnp.int16) ^ jnp.int16(-32768)
    key = jnp.where(in_band, lo_s, jnp.int16(-32768))
    t_lo = jnp.zeros((rows, 1), jnp.int32)
    for bit in range(15, 11, -1):
        cand = t_lo | (1 << bit)
        cand_s16 = (cand - 32768).astype(jnp.int16)
        ind = (key >= cand_s16).astype(jnp.int16)
        t_lo = jnp.where(count16(ind) >= m, cand, t_lo)
    t = (t_hi << 16) | t_lo
    o_ref[...] = jnp.where(abits >= t, hv, 0.0)


def kernel(x, eigvals, eigvecs):
    B, D = x.shape
    C, EIG, _ = eigvecs.shape
    k = min(_TOPK, EIG)
    rb = min(_RB, B)
    grid = (C, B // rb)
    out = pl.pallas_call(
        functools.partial(_select_body, k=k),
        grid=grid,
        in_specs=[
            pl.BlockSpec((rb, D), lambda c, b: (b, 0)),
            pl.BlockSpec((1, EIG, D), lambda c, b: (c, 0, 0)),
            pl.BlockSpec((1, 1, EIG), lambda c, b: (c, 0, 0)),
        ],
        out_specs=pl.BlockSpec((rb, EIG), lambda c, b: (b, c)),
        out_shape=jax.ShapeDtypeStruct((B, C * EIG), jnp.float32),
    )(x, eigvecs, eigvals.reshape(C, 1, EIG))
    return out.reshape(B, C, EIG)
